# Initial kernel scaffold; baseline (speedup 1.0000x reference)
import jax
import jax.numpy as jnp
from jax.experimental import pallas as pl


def kernel(atomic_numbers, edge_index, embedding_weight):
    raise NotImplementedError("write your pallas kernel here")


# speedup vs baseline: 6.6243x; 6.6243x over previous
import functools

import jax
import jax.numpy as jnp
from jax import lax
from jax.experimental import pallas as pl
from jax.experimental.pallas import tpu as pltpu
from jax.experimental.pallas import tpu_sc as plsc

_CHANNELS = 16
_SCALAR_MAX = 4
_BASIS = 9
_OUT_W = 2 * _BASIS * _CHANNELS
_NPAIR = 81

_AN_IDX = jnp.array([0, 0, 0, 0, 0, 0, 1, 2, 3], jnp.int32)
_AN_VALID = jnp.array([False, True, False, False, False, False, True, True, True])
_SDIMS = jnp.array([3, 4, 4, 4], jnp.int32)

_C = 128


def _build_table(w):
    ia = _AN_IDX[:, None]
    ib = _AN_IDX[None, :]
    valid = _AN_VALID[:, None] & _AN_VALID[None, :]
    sfa = w[ia * 4 + ib].reshape(9, 9, _SCALAR_MAX, _CHANNELS)
    sfb = w[ib * 4 + ia].reshape(9, 9, _SCALAR_MAX, _CHANNELS)
    pad = ((0, 0), (0, 0), (0, _BASIS - _SCALAR_MAX), (0, 0))
    sfa_p = jnp.pad(sfa, pad)
    sfb_p = jnp.pad(sfb, pad)
    rows = jnp.arange(_BASIS)[None, None, :, None]
    ma = valid[:, :, None, None] & (rows < _SDIMS[ia][:, :, None, None])
    mb = valid[:, :, None, None] & (rows < _SDIMS[ib][:, :, None, None])
    ta = jnp.where(ma, sfa_p, 0.0)
    tb = jnp.where(mb, sfb_p, 0.0)
    return jnp.concatenate([ta, tb], axis=-1).reshape(_NPAIR, _OUT_W)


def _sc_kernel(num_chunks, num_workers, n_atoms, e_total):
    mesh = plsc.VectorSubcoreMesh(core_axis_name="c", subcore_axis_name="s")

    @functools.partial(
        pl.kernel,
        mesh=mesh,
        compiler_params=pltpu.CompilerParams(use_tc_tiling_on_sc=False),
        out_type=jax.ShapeDtypeStruct((e_total, _OUT_W), jnp.float32),
        scratch_types=[
            pltpu.VMEM((_C,), jnp.int32),
            pltpu.VMEM((_C,), jnp.int32),
            pltpu.VMEM((_C,), jnp.int32),
            pltpu.VMEM((_C,), jnp.int32),
            pltpu.VMEM((_C,), jnp.int32),
            pltpu.VMEM((_C, _OUT_W), jnp.float32),
            pltpu.SemaphoreType.DMA,
        ],
    )
    def body(an_hbm, eidx_hbm, table_hbm, out_hbm,
             i0_v, i1_v, a0_v, a1_v, pair_v, rows_v, sem):
        wid = lax.axis_index("s") * 2 + lax.axis_index("c")

        def chunk_body(t, carry):
            cid = t * num_workers + wid

            @pl.when(cid < num_chunks)
            def _():
                base = cid * _C
                pltpu.sync_copy(eidx_hbm.at[0, pl.ds(base, _C)], i0_v)
                pltpu.sync_copy(eidx_hbm.at[1, pl.ds(base, _C)], i1_v)
                pltpu.async_copy(an_hbm.at[i0_v], a0_v, sem).wait()
                pltpu.async_copy(an_hbm.at[i1_v], a1_v, sem).wait()
                for g in range(_C // 16):
                    sl = pl.ds(g * 16, 16)
                    pair_v[sl] = a0_v[sl] * 9 + a1_v[sl]
                pltpu.async_copy(table_hbm.at[pair_v], rows_v, sem).wait()
                pltpu.sync_copy(rows_v, out_hbm.at[pl.ds(base, _C)])

            return carry

        lax.fori_loop(0, pl.cdiv(num_chunks, num_workers), chunk_body, 0)

    return body


def kernel(atomic_numbers, edge_index, embedding_weight):
    n_atoms = atomic_numbers.shape[0]
    e_total = edge_index.shape[1]
    if e_total % _C != 0:
        raise ValueError("edge count must be a multiple of the chunk size")
    num_chunks = e_total // _C
    table = _build_table(embedding_weight)
    info = plsc.get_sparse_core_info()
    num_workers = info.num_cores * info.num_subcores
    out = _sc_kernel(num_chunks, num_workers, n_atoms, e_total)(
        atomic_numbers, edge_index, table)
    return (out.reshape(e_total, _BASIS, 2 * _CHANNELS), edge_index)
